# speedup vs baseline: 3.0975x; 3.0975x over previous
import jax
import jax.numpy as jnp
from jax.experimental import pallas as pl
from jax.experimental.pallas import tpu as pltpu
from jax.experimental.pallas import tpu_sc as plsc

_WINDOW = 128


def _gather_rows(weight, idx2d, n):
    d = weight.shape[1]
    mesh = plsc.VectorSubcoreMesh(core_axis_name="core", subcore_axis_name="subcore")

    @pl.kernel(
        out_type=jax.ShapeDtypeStruct((n, d), weight.dtype),
        mesh=mesh,
    )
    def k(x_hbm, i_hbm, o_hbm):
        def body(i_vmem, o_vmem):
            pltpu.sync_copy(x_hbm.at[i_vmem.at[0]], o_vmem)

        pltpu.emit_pipeline(
            body,
            grid=(n // _WINDOW,),
            in_specs=[pl.BlockSpec((1, _WINDOW), index_map=lambda i: (0, i))],
            out_specs=[pl.BlockSpec((_WINDOW, d), index_map=lambda i: (i, 0))],
            core_axis_name=("core", "subcore"),
            dimension_semantics=(pltpu.PARALLEL,),
        )(i_hbm, o_hbm)

    return k(weight, idx2d)


def kernel(input_, weight):
    b, s = input_.shape
    n = b * s
    idx2d = input_.reshape(1, n).astype(jnp.int32)
    out = _gather_rows(weight, idx2d, n)
    return out.reshape(b, s, weight.shape[1])


# speedup vs baseline: 3.3387x; 1.0778x over previous
import jax
import jax.numpy as jnp
from jax import lax
from jax.experimental import pallas as pl
from jax.experimental.pallas import tpu as pltpu
from jax.experimental.pallas import tpu_sc as plsc

_CH = 128
_NBUF = 5
_LA = 3
_NW = 32


def _gather_rows(weight, idx2d, n):
    d = weight.shape[1]
    nchunk_total = n // _CH
    nchunk = nchunk_total // _NW
    mesh = plsc.VectorSubcoreMesh(core_axis_name="core", subcore_axis_name="subcore")

    scratch = [pltpu.VMEM((nchunk, _CH), jnp.int32)]
    scratch += [pltpu.VMEM((_CH, d), weight.dtype) for _ in range(_NBUF)]
    scratch += [pltpu.SemaphoreType.DMA for _ in range(2 * _NBUF)]

    @pl.kernel(
        out_type=jax.ShapeDtypeStruct((n, d), weight.dtype),
        mesh=mesh,
        scratch_types=scratch,
    )
    def k(x_hbm, i_hbm, o_hbm, idx_v, *bufs_and_sems):
        bufs = bufs_and_sems[:_NBUF]
        gsem = bufs_and_sems[_NBUF:2 * _NBUF]
        wsem = bufs_and_sems[2 * _NBUF:]

        wid = lax.axis_index("subcore") * 2 + lax.axis_index("core")
        chunk0 = wid * nchunk

        pltpu.sync_copy(i_hbm.at[wid], idx_v)

        def start_gather(h, b):
            pltpu.async_copy(x_hbm.at[idx_v.at[h]], bufs[b], gsem[b])

        def wait_gather(b):
            pltpu.make_async_copy(x_hbm.at[idx_v.at[0]], bufs[b], gsem[b]).wait()

        def start_write(g, b):
            pltpu.async_copy(bufs[b], o_hbm.at[pl.ds((chunk0 + g) * _CH, _CH)], wsem[b])

        def wait_write(b):
            pltpu.make_async_copy(bufs[b], o_hbm.at[pl.ds(0, _CH)], wsem[b]).wait()

        for j in range(_LA):
            start_gather(j, j)

        for b in range(_NBUF):
            h = b + _LA
            hb = h % _NBUF
            if h >= _NBUF:
                wait_write(hb)
                start_gather(h, hb)
            else:
                start_gather(h, hb)
            wait_gather(b)
            start_write(b, b)

        @pl.loop(_NBUF, nchunk, step=_NBUF)
        def _(g0):
            for b in range(_NBUF):
                g = g0 + b
                h = g + _LA
                hb = (b + _LA) % _NBUF

                @pl.when(h < nchunk)
                def _():
                    wait_write(hb)
                    start_gather(h, hb)

                wait_gather(b)
                start_write(g, b)

        for b in range(_NBUF):
            wait_write(b)

    return k(weight, idx2d)


def kernel(input_, weight):
    b, s = input_.shape
    n = b * s
    idx2d = input_.reshape(_NW, n // (_NW * _CH), _CH).astype(jnp.int32)
    out = _gather_rows(weight, idx2d, n)
    return out.reshape(b, s, weight.shape[1])


# speedup vs baseline: 5.9530x; 1.7830x over previous
import jax
import jax.numpy as jnp
from jax import lax
from jax.experimental import pallas as pl
from jax.experimental.pallas import tpu as pltpu
from jax.experimental.pallas import tpu_sc as plsc

_NBUF = 8
_LA = 4
_NW = 32


def _embedding_lookup(weight, idx, b, s):
    d = weight.shape[1]
    per_w = b // _NW
    mesh = plsc.VectorSubcoreMesh(core_axis_name="core", subcore_axis_name="subcore")

    scratch = [pltpu.VMEM((per_w, s), jnp.int32)]
    scratch += [pltpu.VMEM((s, d), weight.dtype) for _ in range(_NBUF)]
    scratch += [pltpu.SemaphoreType.DMA for _ in range(2 * _NBUF)]

    @pl.kernel(
        out_type=jax.ShapeDtypeStruct((b, s, d), weight.dtype),
        mesh=mesh,
        scratch_types=scratch,
    )
    def k(x_hbm, i_hbm, o_hbm, idx_v, *bufs_and_sems):
        bufs = bufs_and_sems[:_NBUF]
        gsem = bufs_and_sems[_NBUF:2 * _NBUF]
        wsem = bufs_and_sems[2 * _NBUF:]

        wid = lax.axis_index("subcore") * 2 + lax.axis_index("core")
        elem0 = wid * per_w

        pltpu.sync_copy(i_hbm.at[pl.ds(elem0, per_w)], idx_v)

        def start_gather(h, bb):
            pltpu.async_copy(x_hbm.at[idx_v.at[h]], bufs[bb], gsem[bb])

        def wait_gather(bb):
            pltpu.make_async_copy(x_hbm.at[idx_v.at[0]], bufs[bb], gsem[bb]).wait()

        def start_write(g, bb):
            pltpu.async_copy(bufs[bb], o_hbm.at[elem0 + g], wsem[bb])

        def wait_write(bb):
            pltpu.make_async_copy(bufs[bb], o_hbm.at[0], wsem[bb]).wait()

        for j in range(_LA):
            start_gather(j, j)

        for bb in range(_NBUF):
            h = bb + _LA
            hb = h % _NBUF
            if h >= _NBUF:
                wait_write(hb)
            start_gather(h, hb)
            wait_gather(bb)
            start_write(bb, bb)

        @pl.loop(_NBUF, per_w, step=_NBUF)
        def _(g0):
            for bb in range(_NBUF):
                g = g0 + bb
                h = g + _LA
                hb = (bb + _LA) % _NBUF

                @pl.when(h < per_w)
                def _():
                    wait_write(hb)
                    start_gather(h, hb)

                wait_gather(bb)
                start_write(g, bb)

        for bb in range(_NBUF):
            wait_write(bb)

    return k(weight, idx)


def kernel(input_, weight):
    b, s = input_.shape
    return _embedding_lookup(weight, input_.astype(jnp.int32), b, s)


# speedup vs baseline: 5.9638x; 1.0018x over previous
import jax
import jax.numpy as jnp
from jax import lax
from jax.experimental import pallas as pl
from jax.experimental.pallas import tpu as pltpu
from jax.experimental.pallas import tpu_sc as plsc

_NBUF = 8
_LA = 4
_NW = 32


def _embedding_lookup(weight, idx, b, s):
    d = weight.shape[1]
    per_w = b // _NW
    mesh = plsc.VectorSubcoreMesh(core_axis_name="core", subcore_axis_name="subcore")

    scratch = [pltpu.VMEM((per_w, s), jnp.int32)]
    scratch += [pltpu.VMEM((s, d), weight.dtype) for _ in range(_NBUF)]
    scratch += [pltpu.SemaphoreType.DMA for _ in range(2 * _NBUF)]

    @pl.kernel(
        out_type=jax.ShapeDtypeStruct((b, s, d), weight.dtype),
        mesh=mesh,
        scratch_types=scratch,
        compiler_params=pltpu.CompilerParams(use_tc_tiling_on_sc=True),
    )
    def k(x_hbm, i_hbm, o_hbm, idx_v, *bufs_and_sems):
        bufs = bufs_and_sems[:_NBUF]
        gsem = bufs_and_sems[_NBUF:2 * _NBUF]
        wsem = bufs_and_sems[2 * _NBUF:]

        wid = lax.axis_index("subcore") * 2 + lax.axis_index("core")
        elem0 = wid * per_w

        pltpu.sync_copy(i_hbm.at[pl.ds(elem0, per_w)], idx_v)

        def start_gather(h, bb):
            pltpu.async_copy(x_hbm.at[idx_v.at[h]], bufs[bb], gsem[bb])

        def wait_gather(bb):
            pltpu.make_async_copy(x_hbm.at[idx_v.at[0]], bufs[bb], gsem[bb]).wait()

        def start_write(g, bb):
            pltpu.async_copy(bufs[bb], o_hbm.at[elem0 + g], wsem[bb])

        def wait_write(bb):
            pltpu.make_async_copy(bufs[bb], o_hbm.at[0], wsem[bb]).wait()

        for j in range(_LA):
            start_gather(j, j)

        for bb in range(_NBUF):
            h = bb + _LA
            hb = h % _NBUF
            if h >= _NBUF:
                wait_write(hb)
            start_gather(h, hb)
            wait_gather(bb)
            start_write(bb, bb)

        @pl.loop(_NBUF, per_w, step=_NBUF)
        def _(g0):
            for bb in range(_NBUF):
                g = g0 + bb
                h = g + _LA
                hb = (bb + _LA) % _NBUF

                @pl.when(h < per_w)
                def _():
                    wait_write(hb)
                    start_gather(h, hb)

                wait_gather(bb)
                start_write(g, bb)

        for bb in range(_NBUF):
            wait_write(bb)

    return k(weight, idx)


def kernel(input_, weight):
    b, s = input_.shape
    return _embedding_lookup(weight, input_.astype(jnp.int32), b, s)


# speedup vs baseline: 10.6650x; 1.7883x over previous
import jax
import jax.numpy as jnp
from jax import lax
from jax.experimental import pallas as pl
from jax.experimental.pallas import tpu as pltpu
from jax.experimental.pallas import tpu_sc as plsc

_BB = 128
_NBUF = 5
_LA = 3
_NW = 32


def _embedding_lookup(weight, idx_t, b, s):
    d = weight.shape[1]
    mesh = plsc.VectorSubcoreMesh(core_axis_name="core", subcore_axis_name="subcore")

    scratch = [pltpu.VMEM((s, _BB), jnp.int32)]
    scratch += [pltpu.VMEM((_BB, d), weight.dtype) for _ in range(_NBUF)]
    scratch += [pltpu.SemaphoreType.DMA for _ in range(2 * _NBUF)]

    @pl.kernel(
        out_type=jax.ShapeDtypeStruct((s, b, d), weight.dtype),
        mesh=mesh,
        scratch_types=scratch,
    )
    def k(x_hbm, i_hbm, o_hbm, idx_v, *bufs_and_sems):
        bufs = bufs_and_sems[:_NBUF]
        gsem = bufs_and_sems[_NBUF:2 * _NBUF]
        wsem = bufs_and_sems[2 * _NBUF:]

        wid = lax.axis_index("subcore") * 2 + lax.axis_index("core")
        b0 = wid * _BB

        pltpu.sync_copy(i_hbm.at[:, pl.ds(b0, _BB)], idx_v)

        def start_gather(h, bb):
            pltpu.async_copy(x_hbm.at[idx_v.at[h]], bufs[bb], gsem[bb])

        def wait_gather(bb):
            pltpu.make_async_copy(x_hbm.at[idx_v.at[0]], bufs[bb], gsem[bb]).wait()

        def start_write(g, bb):
            pltpu.async_copy(bufs[bb], o_hbm.at[g, pl.ds(b0, _BB)], wsem[bb])

        def wait_write(bb):
            pltpu.make_async_copy(bufs[bb], o_hbm.at[0, pl.ds(b0, _BB)], wsem[bb]).wait()

        for j in range(_LA):
            start_gather(j, j)

        for bb in range(_NBUF):
            h = bb + _LA
            hb = h % _NBUF
            if h >= _NBUF:
                wait_write(hb)
            start_gather(h, hb)
            wait_gather(bb)
            start_write(bb, bb)

        @pl.loop(_NBUF, s, step=_NBUF)
        def _(g0):
            for bb in range(_NBUF):
                g = g0 + bb
                h = g + _LA
                hb = (bb + _LA) % _NBUF

                @pl.when(h < s)
                def _():
                    wait_write(hb)
                    start_gather(h, hb)

                wait_gather(bb)
                start_write(g, bb)

        for bb in range(_NBUF):
            wait_write(bb)

    return k(weight, idx_t)


def kernel(input_, weight):
    b, s = input_.shape
    idx_t = jnp.transpose(input_.astype(jnp.int32))
    out = _embedding_lookup(weight, idx_t, b, s)
    return jnp.transpose(out, (1, 0, 2))


# speedup vs baseline: 10.7067x; 1.0039x over previous
import jax
import jax.numpy as jnp
from jax import lax
from jax.experimental import pallas as pl
from jax.experimental.pallas import tpu as pltpu
from jax.experimental.pallas import tpu_sc as plsc

_BB = 128
_NBUF = 5
_LA = 4
_NW = 32


def _embedding_lookup(weight, idx_t, b, s):
    d = weight.shape[1]
    mesh = plsc.VectorSubcoreMesh(core_axis_name="core", subcore_axis_name="subcore")

    scratch = [pltpu.VMEM((s, _BB), jnp.int32)]
    scratch += [pltpu.VMEM((_BB, d), weight.dtype) for _ in range(_NBUF)]
    scratch += [pltpu.SemaphoreType.DMA for _ in range(2 * _NBUF)]

    @pl.kernel(
        out_type=jax.ShapeDtypeStruct((s, b, d), weight.dtype),
        mesh=mesh,
        scratch_types=scratch,
    )
    def k(x_hbm, i_hbm, o_hbm, idx_v, *bufs_and_sems):
        bufs = bufs_and_sems[:_NBUF]
        gsem = bufs_and_sems[_NBUF:2 * _NBUF]
        wsem = bufs_and_sems[2 * _NBUF:]

        wid = lax.axis_index("subcore") * 2 + lax.axis_index("core")
        b0 = wid * _BB

        pltpu.sync_copy(i_hbm.at[:, pl.ds(b0, _BB)], idx_v)

        def start_gather(h, bb):
            pltpu.async_copy(x_hbm.at[idx_v.at[h]], bufs[bb], gsem[bb])

        def wait_gather(bb):
            pltpu.make_async_copy(x_hbm.at[idx_v.at[0]], bufs[bb], gsem[bb]).wait()

        def start_write(g, bb):
            pltpu.async_copy(bufs[bb], o_hbm.at[g, pl.ds(b0, _BB)], wsem[bb])

        def wait_write(bb):
            pltpu.make_async_copy(bufs[bb], o_hbm.at[0, pl.ds(b0, _BB)], wsem[bb]).wait()

        for j in range(_LA):
            start_gather(j, j)

        for bb in range(_NBUF):
            h = bb + _LA
            hb = h % _NBUF
            if h >= _NBUF:
                wait_write(hb)
            start_gather(h, hb)
            wait_gather(bb)
            start_write(bb, bb)

        @pl.loop(_NBUF, s, step=_NBUF)
        def _(g0):
            for bb in range(_NBUF):
                g = g0 + bb
                h = g + _LA
                hb = (bb + _LA) % _NBUF

                @pl.when(h < s)
                def _():
                    wait_write(hb)
                    start_gather(h, hb)

                wait_gather(bb)
                start_write(g, bb)

        for bb in range(_NBUF):
            wait_write(bb)

    return k(weight, idx_t)


def kernel(input_, weight):
    b, s = input_.shape
    idx_t = jnp.transpose(input_.astype(jnp.int32))
    out = _embedding_lookup(weight, idx_t, b, s)
    return jnp.transpose(out, (1, 0, 2))


# speedup vs baseline: 10.7288x; 1.0021x over previous
import jax
import jax.numpy as jnp
from jax import lax
from jax.experimental import pallas as pl
from jax.experimental.pallas import tpu as pltpu
from jax.experimental.pallas import tpu_sc as plsc

_BB = 128
_NBUF = 7
_LA = 6
_NW = 32


def _embedding_lookup(weight, idx_t, b, s):
    d = weight.shape[1]
    mesh = plsc.VectorSubcoreMesh(core_axis_name="core", subcore_axis_name="subcore")

    scratch = [pltpu.VMEM((s, _BB), jnp.int32)]
    scratch += [pltpu.VMEM((_BB, d), weight.dtype) for _ in range(_NBUF)]
    scratch += [pltpu.SemaphoreType.DMA for _ in range(2 * _NBUF)]

    @pl.kernel(
        out_type=jax.ShapeDtypeStruct((s, b, d), weight.dtype),
        mesh=mesh,
        scratch_types=scratch,
    )
    def k(x_hbm, i_hbm, o_hbm, idx_v, *bufs_and_sems):
        bufs = bufs_and_sems[:_NBUF]
        gsem = bufs_and_sems[_NBUF:2 * _NBUF]
        wsem = bufs_and_sems[2 * _NBUF:]

        wid = lax.axis_index("subcore") * 2 + lax.axis_index("core")
        b0 = wid * _BB

        pltpu.sync_copy(i_hbm.at[:, pl.ds(b0, _BB)], idx_v)

        def start_gather(h, bb):
            pltpu.async_copy(x_hbm.at[idx_v.at[h]], bufs[bb], gsem[bb])

        def wait_gather(bb):
            pltpu.make_async_copy(x_hbm.at[idx_v.at[0]], bufs[bb], gsem[bb]).wait()

        def start_write(g, bb):
            pltpu.async_copy(bufs[bb], o_hbm.at[g, pl.ds(b0, _BB)], wsem[bb])

        def wait_write(bb):
            pltpu.make_async_copy(bufs[bb], o_hbm.at[0, pl.ds(b0, _BB)], wsem[bb]).wait()

        for j in range(_LA):
            start_gather(j, j)

        for bb in range(_NBUF):
            h = bb + _LA
            hb = h % _NBUF
            if h >= _NBUF:
                wait_write(hb)
            start_gather(h, hb)
            wait_gather(bb)
            start_write(bb, bb)

        @pl.loop(_NBUF, s, step=_NBUF)
        def _(g0):
            for bb in range(_NBUF):
                g = g0 + bb
                h = g + _LA
                hb = (bb + _LA) % _NBUF

                @pl.when(h < s)
                def _():
                    wait_write(hb)
                    start_gather(h, hb)

                @pl.when(g < s)
                def _():
                    wait_gather(bb)
                    start_write(g, bb)

        for bb in range(_NBUF):
            wait_write(bb)

    return k(weight, idx_t)


def kernel(input_, weight):
    b, s = input_.shape
    idx_t = jnp.transpose(input_.astype(jnp.int32))
    out = _embedding_lookup(weight, idx_t, b, s)
    return jnp.transpose(out, (1, 0, 2))


# speedup vs baseline: 10.7521x; 1.0022x over previous
import jax
import jax.numpy as jnp
from jax import lax
from jax.experimental import pallas as pl
from jax.experimental.pallas import tpu as pltpu
from jax.experimental.pallas import tpu_sc as plsc

_BB = 128
_NBUF = 7
_LA = 6


def _embedding_lookup(weight, idx_t, b, s):
    d = weight.shape[1]
    mesh = plsc.VectorSubcoreMesh(core_axis_name="core", subcore_axis_name="subcore")

    scratch = [pltpu.VMEM((s, _BB), jnp.int32)]
    scratch += [pltpu.VMEM((_BB, d), weight.dtype) for _ in range(_NBUF)]
    scratch += [pltpu.SemaphoreType.DMA for _ in range(2 * _NBUF)]

    @pl.kernel(
        out_type=jax.ShapeDtypeStruct((s, b, d), weight.dtype),
        mesh=mesh,
        scratch_types=scratch,
    )
    def k(x_hbm, i_hbm, o_hbm, idx_v, *bufs_and_sems):
        bufs = bufs_and_sems[:_NBUF]
        gsem = bufs_and_sems[_NBUF:2 * _NBUF]
        wsem = bufs_and_sems[2 * _NBUF:]

        wid = lax.axis_index("subcore") * 2 + lax.axis_index("core")
        b0 = wid * _BB

        pltpu.sync_copy(i_hbm.at[:, pl.ds(b0, _BB)], idx_v)

        def start_gather(h, bb):
            pltpu.async_copy(x_hbm.at[idx_v.at[h]], bufs[bb], gsem[bb])

        def wait_gather(bb):
            pltpu.make_async_copy(x_hbm.at[idx_v.at[0]], bufs[bb], gsem[bb]).wait()

        def start_write(g, bb):
            pltpu.async_copy(bufs[bb], o_hbm.at[g, pl.ds(b0, _BB)], wsem[bb])

        def wait_write(bb):
            pltpu.make_async_copy(bufs[bb], o_hbm.at[0, pl.ds(b0, _BB)], wsem[bb]).wait()

        for j in range(_LA):
            start_gather(j, j)

        for bb in range(_NBUF):
            h = bb + _LA
            hb = h % _NBUF
            if h >= _NBUF:
                wait_write(hb)
            start_gather(h, hb)
            wait_gather(bb)
            start_write(bb, bb)

        @pl.loop(_NBUF, s, step=_NBUF)
        def _(g0):
            for bb in range(_NBUF):
                g = g0 + bb
                h = g + _LA
                hb = (bb + _LA) % _NBUF

                @pl.when(h < s)
                def _():
                    wait_write(hb)
                    start_gather(h, hb)

                @pl.when(g < s)
                def _():
                    wait_gather(bb)
                    start_write(g, bb)

        for bb in range(_NBUF):
            wait_write(bb)

    return k(weight, idx_t)


def kernel(input_, weight):
    b, s = input_.shape
    idx_t = jnp.transpose(input_.astype(jnp.int32))
    out = _embedding_lookup(weight, idx_t, b, s)
    return jnp.transpose(out, (1, 0, 2))
